# Initial kernel scaffold; baseline (speedup 1.0000x reference)
import jax
import jax.numpy as jnp
from jax.experimental import pallas as pl


def kernel(v_idx, d_idx, j_idx, v_bank, d_bank, j_bank):
    raise NotImplementedError("write your pallas kernel here")


# speedup vs baseline: 2.3215x; 2.3215x over previous
import functools

import jax
import jax.numpy as jnp
from jax import lax
from jax.experimental import pallas as pl
from jax.experimental.pallas import tpu as pltpu
from jax.experimental.pallas import tpu_sc as plsc

_OUT_D = 128
_SEG = 42
_B = 16384
_NC, _NS = 2, 16
_NW = _NC * _NS
_BPW = _B // _NW
_CHUNK = 128
_NCHUNK = _BPW // _CHUNK
_PW = 48
_D_OFF = 40
_J_OFF = 80

_mesh = plsc.VectorSubcoreMesh(core_axis_name="c", subcore_axis_name="s")


@functools.partial(
    pl.kernel,
    mesh=_mesh,
    compiler_params=pltpu.CompilerParams(use_tc_tiling_on_sc=False),
    out_type=jax.ShapeDtypeStruct((_B, _OUT_D), jnp.float32),
    scratch_types=[
        pltpu.VMEM((_NCHUNK, _CHUNK), jnp.int32),
        pltpu.VMEM((_NCHUNK, _CHUNK), jnp.int32),
        pltpu.VMEM((_NCHUNK, _CHUNK), jnp.int32),
        pltpu.VMEM((_BPW, _OUT_D), jnp.float32),
        pltpu.VMEM((_CHUNK, _PW), jnp.float32),
        pltpu.VMEM((_CHUNK, _PW), jnp.float32),
        pltpu.SemaphoreType.DMA,
    ],
)
def _recombine(v_idx_hbm, d_idx_hbm, j_idx_hbm,
               v_bank_hbm, d_bank_hbm, j_bank_hbm,
               out_hbm, vi, di, ji, rows, d_scr, j_scr, sem):
    wid = lax.axis_index("s") * _NC + lax.axis_index("c")
    base = wid * _BPW

    pltpu.sync_copy(v_idx_hbm.at[pl.ds(wid * _NCHUNK, _NCHUNK)], vi)
    pltpu.sync_copy(d_idx_hbm.at[pl.ds(wid * _NCHUNK, _NCHUNK)], di)
    pltpu.sync_copy(j_idx_hbm.at[pl.ds(wid * _NCHUNK, _NCHUNK)], ji)

    for c in range(_NCHUNK):
        rsl = pl.ds(c * _CHUNK, _CHUNK)
        cp_v = pltpu.async_copy(v_bank_hbm.at[vi.at[c]], rows.at[rsl], sem)
        cp_d = pltpu.async_copy(d_bank_hbm.at[di.at[c]], d_scr, sem)
        cp_j = pltpu.async_copy(j_bank_hbm.at[ji.at[c]], j_scr, sem)
        cp_v.wait()
        cp_d.wait()
        cp_j.wait()

        def row_body(r, carry, c=c):
            rr = c * _CHUNK + r
            for t in range(3):
                off = _D_OFF + 16 * t
                rows[rr, pl.ds(off, 16)] = (
                    rows[rr, pl.ds(off, 16)] + d_scr[r, pl.ds(16 * t, 16)])
            for t in range(3):
                off = _J_OFF + 16 * t
                rows[rr, pl.ds(off, 16)] = (
                    rows[rr, pl.ds(off, 16)] + j_scr[r, pl.ds(16 * t, 16)])
            return carry

        lax.fori_loop(0, _CHUNK, row_body, 0)

    pltpu.sync_copy(rows, out_hbm.at[pl.ds(base, _BPW)])


def kernel(v_idx, d_idx, j_idx, v_bank, d_bank, j_bank):
    vi = v_idx.astype(jnp.int32).reshape(_NW * _NCHUNK, _CHUNK)
    di = d_idx.astype(jnp.int32).reshape(_NW * _NCHUNK, _CHUNK)
    ji = j_idx.astype(jnp.int32).reshape(_NW * _NCHUNK, _CHUNK)
    v_p = jnp.pad(v_bank, ((0, 0), (0, _OUT_D - _SEG)))
    d_p = jnp.pad(d_bank, ((0, 0), (_SEG - _D_OFF, _PW - _SEG - (_SEG - _D_OFF))))
    j_p = jnp.pad(j_bank, ((0, 0), (2 * _SEG - _J_OFF, 0)))
    return _recombine(vi, di, ji, v_p, d_p, j_p)


# speedup vs baseline: 2.5660x; 1.1053x over previous
import functools

import jax
import jax.numpy as jnp
from jax import lax
from jax.experimental import pallas as pl
from jax.experimental.pallas import tpu as pltpu
from jax.experimental.pallas import tpu_sc as plsc

_OUT_D = 128
_SEG = 42
_B = 16384
_NC, _NS = 2, 16
_NW = _NC * _NS
_BPW = _B // _NW
_CHUNK = 128
_NCHUNK = _BPW // _CHUNK
_SW = 48

_mesh = plsc.VectorSubcoreMesh(core_axis_name="c", subcore_axis_name="s")


@functools.partial(
    pl.kernel,
    mesh=_mesh,
    compiler_params=pltpu.CompilerParams(use_tc_tiling_on_sc=False),
    out_type=jax.ShapeDtypeStruct((_B, _OUT_D), jnp.float32),
    scratch_types=[
        pltpu.VMEM((_NCHUNK, _CHUNK), jnp.int32),
        pltpu.VMEM((_NCHUNK, _CHUNK), jnp.int32),
        pltpu.VMEM((_NCHUNK, _CHUNK), jnp.int32),
        pltpu.VMEM((_BPW, _OUT_D), jnp.float32),
        pltpu.VMEM((_BPW, _SW), jnp.float32),
        pltpu.VMEM((_BPW, _SW), jnp.float32),
        pltpu.SemaphoreType.DMA,
        pltpu.SemaphoreType.DMA,
    ],
)
def _recombine(v_idx_hbm, d_idx_hbm, j_idx_hbm,
               v_bank_hbm, d_bank_hbm, j_bank_hbm,
               out_hbm, vi, di, ji, rows, v_scr, j_scr, sem_i, sem_g):
    wid = lax.axis_index("s") * _NC + lax.axis_index("c")
    base = wid * _BPW

    cpi = [
        pltpu.async_copy(v_idx_hbm.at[pl.ds(wid * _NCHUNK, _NCHUNK)], vi, sem_i),
        pltpu.async_copy(d_idx_hbm.at[pl.ds(wid * _NCHUNK, _NCHUNK)], di, sem_i),
        pltpu.async_copy(j_idx_hbm.at[pl.ds(wid * _NCHUNK, _NCHUNK)], ji, sem_i),
    ]
    for cp in cpi:
        cp.wait()

    cps = []
    for c in range(_NCHUNK):
        rsl = pl.ds(c * _CHUNK, _CHUNK)
        cps.append(pltpu.async_copy(d_bank_hbm.at[di.at[c]], rows.at[rsl], sem_g))
        cps.append(pltpu.async_copy(v_bank_hbm.at[vi.at[c]], v_scr.at[rsl], sem_g))
        cps.append(pltpu.async_copy(j_bank_hbm.at[ji.at[c]], j_scr.at[rsl], sem_g))
    for cp in cps:
        cp.wait()

    def row_body(r, carry):
        rows[r, pl.ds(0, 16)] = v_scr[r, pl.ds(0, 16)]
        rows[r, pl.ds(16, 16)] = v_scr[r, pl.ds(16, 16)]
        rows[r, pl.ds(32, 16)] = rows[r, pl.ds(32, 16)] + v_scr[r, pl.ds(32, 16)]
        rows[r, pl.ds(80, 16)] = rows[r, pl.ds(80, 16)] + j_scr[r, pl.ds(0, 16)]
        rows[r, pl.ds(96, 16)] = j_scr[r, pl.ds(16, 16)]
        rows[r, pl.ds(112, 16)] = j_scr[r, pl.ds(32, 16)]
        return carry

    lax.fori_loop(0, _BPW, row_body, 0)

    pltpu.sync_copy(rows, out_hbm.at[pl.ds(base, _BPW)])


def kernel(v_idx, d_idx, j_idx, v_bank, d_bank, j_bank):
    vi = v_idx.astype(jnp.int32).reshape(_NW * _NCHUNK, _CHUNK)
    di = d_idx.astype(jnp.int32).reshape(_NW * _NCHUNK, _CHUNK)
    ji = j_idx.astype(jnp.int32).reshape(_NW * _NCHUNK, _CHUNK)
    v_p = jnp.pad(v_bank, ((0, 0), (0, _SW - _SEG)))
    d_p = jnp.pad(d_bank, ((0, 0), (_SEG, _OUT_D - 2 * _SEG)))
    j_p = jnp.pad(j_bank, ((0, 0), (_SW - (_OUT_D - 2 * _SEG), 0)))
    return _recombine(vi, di, ji, v_p, d_p, j_p)


# speedup vs baseline: 4.7531x; 1.8523x over previous
import functools

import jax
import jax.numpy as jnp
from jax import lax
from jax.experimental import pallas as pl
from jax.experimental.pallas import tpu as pltpu
from jax.experimental.pallas import tpu_sc as plsc

_OUT_D = 128
_SEG = 42
_B = 16384
_NC, _NS = 2, 16
_NW = _NC * _NS
_BPW = _B // _NW
_CHUNK = 128
_NCHUNK = _BPW // _CHUNK
_SW = 48
_G = 16

_mesh = plsc.VectorSubcoreMesh(core_axis_name="c", subcore_axis_name="s")


@functools.partial(
    pl.kernel,
    mesh=_mesh,
    compiler_params=pltpu.CompilerParams(use_tc_tiling_on_sc=False),
    out_type=jax.ShapeDtypeStruct((_B, _OUT_D), jnp.float32),
    scratch_types=[
        pltpu.VMEM((_NCHUNK, _CHUNK), jnp.int32),
        pltpu.VMEM((_NCHUNK, _CHUNK), jnp.int32),
        pltpu.VMEM((_NCHUNK, _CHUNK), jnp.int32),
        pltpu.VMEM((64, _SW), jnp.float32),
        pltpu.VMEM((32, _OUT_D), jnp.float32),
        pltpu.VMEM((16, _SW), jnp.float32),
        pltpu.VMEM((_BPW, _OUT_D), jnp.float32),
        pltpu.SemaphoreType.DMA,
        pltpu.SemaphoreType.DMA,
    ],
)
def _recombine(v_idx_hbm, d_idx_hbm, j_idx_hbm,
               v_bank_hbm, d_bank_hbm, j_bank_hbm,
               out_hbm, vi, di, ji, vb, db, jb, rows, sem_i, sem_o):
    wid = lax.axis_index("s") * _NC + lax.axis_index("c")
    base = wid * _BPW

    cpi = [
        pltpu.async_copy(v_idx_hbm.at[pl.ds(wid * _NCHUNK, _NCHUNK)], vi, sem_i),
        pltpu.async_copy(d_idx_hbm.at[pl.ds(wid * _NCHUNK, _NCHUNK)], di, sem_i),
        pltpu.async_copy(j_idx_hbm.at[pl.ds(wid * _NCHUNK, _NCHUNK)], ji, sem_i),
        pltpu.async_copy(v_bank_hbm, vb, sem_i),
        pltpu.async_copy(d_bank_hbm, db, sem_i),
        pltpu.async_copy(j_bank_hbm, jb, sem_i),
    ]
    for cp in cpi:
        cp.wait()

    out_cps = []
    for c in range(_NCHUNK):
        def grp_body(g, carry, c=c):
            o16 = g * _G
            vvec = vi[c, pl.ds(o16, _G)]
            dvec = di[c, pl.ds(o16, _G)]
            jvec = ji[c, pl.ds(o16, _G)]
            for l in range(_G):
                rr = c * _CHUNK + o16 + l
                r_v = vvec[l]
                r_d = dvec[l]
                r_j = jvec[l]
                rows[rr, pl.ds(0, 16)] = vb[r_v, pl.ds(0, 16)]
                rows[rr, pl.ds(16, 16)] = vb[r_v, pl.ds(16, 16)]
                rows[rr, pl.ds(32, 16)] = (vb[r_v, pl.ds(32, 16)]
                                           + db[r_d, pl.ds(32, 16)])
                rows[rr, pl.ds(48, 16)] = db[r_d, pl.ds(48, 16)]
                rows[rr, pl.ds(64, 16)] = db[r_d, pl.ds(64, 16)]
                rows[rr, pl.ds(80, 16)] = (db[r_d, pl.ds(80, 16)]
                                           + jb[r_j, pl.ds(0, 16)])
                rows[rr, pl.ds(96, 16)] = jb[r_j, pl.ds(16, 16)]
                rows[rr, pl.ds(112, 16)] = jb[r_j, pl.ds(32, 16)]
            return carry

        lax.fori_loop(0, _CHUNK // _G, grp_body, 0)
        out_cps.append(pltpu.async_copy(
            rows.at[pl.ds(c * _CHUNK, _CHUNK)],
            out_hbm.at[pl.ds(base + c * _CHUNK, _CHUNK)], sem_o))
    for cp in out_cps:
        cp.wait()


def kernel(v_idx, d_idx, j_idx, v_bank, d_bank, j_bank):
    vi = v_idx.astype(jnp.int32).reshape(_NW * _NCHUNK, _CHUNK)
    di = d_idx.astype(jnp.int32).reshape(_NW * _NCHUNK, _CHUNK)
    ji = j_idx.astype(jnp.int32).reshape(_NW * _NCHUNK, _CHUNK)
    v_p = jnp.pad(v_bank, ((0, 0), (0, _SW - _SEG)))
    d_p = jnp.pad(d_bank, ((0, 0), (_SEG, _OUT_D - 2 * _SEG)))
    j_p = jnp.pad(j_bank, ((0, 0), (_SW - (_OUT_D - 2 * _SEG), 0)))
    return _recombine(vi, di, ji, v_p, d_p, j_p)
